# Initial kernel scaffold; baseline (speedup 1.0000x reference)
import jax
import jax.numpy as jnp
from jax.experimental import pallas as pl


def kernel(x, adj_indices, adj_values, W1, b1, W2, b2, ln_gamma, ln_beta):
    raise NotImplementedError("write your pallas kernel here")


# speedup vs baseline: 3.6625x; 3.6625x over previous
import functools

import jax
import jax.numpy as jnp
from jax import lax
from jax.experimental import pallas as pl
from jax.experimental.pallas import tpu as pltpu
from jax.experimental.pallas import tpu_sc as plsc

_N = 10000
_E = 320000
_D = 128
_H = 128

_NC = 2
_NS = 16
_NW = _NC * _NS
_CHUNK = 128
_EPAD = ((_E + _NW * _CHUNK - 1) // (_NW * _CHUNK)) * (_NW * _CHUNK)
_EPW = _EPAD // _NW
_NCHUNK = _EPW // _CHUNK
_NP = 10240
_RPT = _NP // _NS
_ZR = 128


def _spmm_sc_body(h_hbm, col_hbm, row_hbm, val_hbm, out_hbm,
                  colv, rowv, valv, rows, zbuf, acc, sem):
    cid = lax.axis_index("c")
    sid = lax.axis_index("s")
    wid = sid * _NC + cid

    def _zrow(i, carry):
        for f in range(_H // 16):
            zbuf[i, pl.ds(f * 16, 16)] = jnp.zeros((16,), jnp.float32)
        return carry
    lax.fori_loop(0, _ZR, _zrow, 0)
    for k in range(_RPT // _ZR):
        pltpu.sync_copy(zbuf, acc.at[pl.ds(sid * _RPT + k * _ZR, _ZR)])
    plsc.subcore_barrier()

    ebase = wid * _EPW

    def _chunk(g, carry):
        base = ebase + g * _CHUNK
        pltpu.sync_copy(col_hbm.at[pl.ds(base, _CHUNK)], colv)
        pltpu.sync_copy(row_hbm.at[pl.ds(base, _CHUNK)], rowv)
        pltpu.sync_copy(val_hbm.at[pl.ds(base, _CHUNK)], valv)
        pltpu.async_copy(h_hbm.at[colv], rows, sem).wait()

        def _grp(g16, c2):
            v16 = valv[pl.ds(g16 * 16, 16)]
            for j in range(16):
                e = g16 * 16 + j
                bv = jnp.full((16,), v16[j], jnp.float32)
                for f in range(_H // 16):
                    sl = (e, pl.ds(f * 16, 16))
                    rows[sl] = rows[sl] * bv
            return c2
        lax.fori_loop(0, _CHUNK // 16, _grp, 0)
        pltpu.sync_copy(rows, acc.at[rowv], add=True)
        return carry
    lax.fori_loop(0, _NCHUNK, _chunk, 0)

    plsc.subcore_barrier()
    pltpu.sync_copy(acc.at[pl.ds(sid * _RPT, _RPT)],
                    out_hbm.at[cid, pl.ds(sid * _RPT, _RPT)])


def _spmm_partials(h, col, row, val):
    mesh = plsc.VectorSubcoreMesh(core_axis_name="c", subcore_axis_name="s")
    k = functools.partial(
        pl.kernel, mesh=mesh,
        out_type=jax.ShapeDtypeStruct((_NC, _NP, _H), jnp.float32),
        scratch_types=[
            pltpu.VMEM((_CHUNK,), jnp.int32),
            pltpu.VMEM((_CHUNK,), jnp.int32),
            pltpu.VMEM((_CHUNK,), jnp.float32),
            pltpu.VMEM((_CHUNK, _H), jnp.float32),
            pltpu.VMEM((_ZR, _H), jnp.float32),
            pltpu.VMEM_SHARED((_NP, _H), jnp.float32),
            pltpu.SemaphoreType.DMA,
        ],
    )(_spmm_sc_body)
    return k(h, col, row, val)


def _gelu(x):
    return 0.5 * x * (1.0 + lax.erf(x * (2.0 ** -0.5)))


def _lin1_body(x_ref, w_ref, b_ref, o_ref):
    o_ref[...] = (
        jnp.dot(x_ref[...], w_ref[...], preferred_element_type=jnp.float32)
        + b_ref[...]
    )


def _mid_body(p_ref, w_ref, b_ref, o_ref):
    s = _gelu(p_ref[0] + p_ref[1])
    o_ref[...] = (
        jnp.dot(s, w_ref[...], preferred_element_type=jnp.float32)
        + b_ref[...]
    )


def _fin_body(p_ref, x_ref, g_ref, bt_ref, o_ref):
    r = _gelu(p_ref[0] + p_ref[1]) + x_ref[...]
    mean = jnp.mean(r, axis=-1, keepdims=True)
    var = jnp.mean((r - mean) ** 2, axis=-1, keepdims=True)
    o_ref[...] = (r - mean) / jnp.sqrt(var + 1e-5) * g_ref[...] + bt_ref[...]


_BLK = 1000
_GRID = _N // _BLK


def _linear1(x, w_t, b):
    return pl.pallas_call(
        _lin1_body,
        grid=(_GRID,),
        in_specs=[
            pl.BlockSpec((_BLK, _D), lambda i: (i, 0)),
            pl.BlockSpec((_D, _H), lambda i: (0, 0)),
            pl.BlockSpec((1, _H), lambda i: (0, 0)),
        ],
        out_specs=pl.BlockSpec((_BLK, _H), lambda i: (i, 0)),
        out_shape=jax.ShapeDtypeStruct((_N, _H), jnp.float32),
    )(x, w_t, b)


def _mid(p, w_t, b):
    return pl.pallas_call(
        _mid_body,
        grid=(_GRID,),
        in_specs=[
            pl.BlockSpec((_NC, _BLK, _H), lambda i: (0, i, 0)),
            pl.BlockSpec((_H, _H), lambda i: (0, 0)),
            pl.BlockSpec((1, _H), lambda i: (0, 0)),
        ],
        out_specs=pl.BlockSpec((_BLK, _H), lambda i: (i, 0)),
        out_shape=jax.ShapeDtypeStruct((_N, _H), jnp.float32),
    )(p, w_t, b)


def _final(p, x, gamma, beta):
    return pl.pallas_call(
        _fin_body,
        grid=(_GRID,),
        in_specs=[
            pl.BlockSpec((_NC, _BLK, _H), lambda i: (0, i, 0)),
            pl.BlockSpec((_BLK, _H), lambda i: (i, 0)),
            pl.BlockSpec((1, _H), lambda i: (0, 0)),
            pl.BlockSpec((1, _H), lambda i: (0, 0)),
        ],
        out_specs=pl.BlockSpec((_BLK, _H), lambda i: (i, 0)),
        out_shape=jax.ShapeDtypeStruct((_N, _H), jnp.float32),
    )(p, x, gamma, beta)


def kernel(x, adj_indices, adj_values, W1, b1, W2, b2, ln_gamma, ln_beta):
    pad = _EPAD - _E
    row = jnp.pad(adj_indices[0], (0, pad))
    col = jnp.pad(adj_indices[1], (0, pad))
    val = jnp.pad(adj_values, (0, pad))

    t1 = _linear1(x, W1.T, b1.reshape(1, _H))
    p1 = _spmm_partials(t1, col, row, val)
    t2 = _mid(p1, W2.T, b2.reshape(1, _H))
    p2 = _spmm_partials(t2, col, row, val)
    return _final(p2, x[:, :_H], ln_gamma.reshape(1, _H),
                  ln_beta.reshape(1, _H))


# speedup vs baseline: 3.7794x; 1.0319x over previous
import functools

import jax
import jax.numpy as jnp
from jax import lax
from jax.experimental import pallas as pl
from jax.experimental.pallas import tpu as pltpu
from jax.experimental.pallas import tpu_sc as plsc

_N = 10000
_E = 320000
_D = 128
_H = 128

_NC = 2
_NS = 16
_NW = _NC * _NS
_CHUNK = 80
_NCHUNK = 128
_EPW = _NCHUNK * _CHUNK
_EPAD = _NW * _EPW
_NP = 10240
_RPT = _NP // _NS
_NBUF = 4
_NIDX = 8


def _spmm_sc_body(h_hbm, col_hbm, row_hbm, val_hbm, out_hbm,
                  colv, rowv, valv, b0, b1, b2, b3, acc,
                  g0, g1, g2, g3, s0, s1, s2, s3,
                  i0, i1, i2, i3, i4, i5, i6, i7):
    bufs = (b0, b1, b2, b3)
    gsem = (g0, g1, g2, g3)
    ssem = (s0, s1, s2, s3)
    isem = (i0, i1, i2, i3, i4, i5, i6, i7)
    cid = lax.axis_index("c")
    sid = lax.axis_index("s")
    wid = sid * _NC + cid
    ebase = wid * _EPW

    def _idx_start(g, q):
        off = ebase + g * _CHUNK
        pltpu.async_copy(col_hbm.at[pl.ds(off, _CHUNK)], colv.at[q], isem[q])
        pltpu.async_copy(row_hbm.at[pl.ds(off, _CHUNK)], rowv.at[q], isem[q])
        pltpu.async_copy(val_hbm.at[pl.ds(off, _CHUNK)], valv.at[q], isem[q])

    def _idx_wait(g, q):
        off = ebase + g * _CHUNK
        pltpu.make_async_copy(col_hbm.at[pl.ds(off, _CHUNK)], colv.at[q],
                              isem[q]).wait()
        pltpu.make_async_copy(row_hbm.at[pl.ds(off, _CHUNK)], rowv.at[q],
                              isem[q]).wait()
        pltpu.make_async_copy(val_hbm.at[pl.ds(off, _CHUNK)], valv.at[q],
                              isem[q]).wait()

    def _gather_start(q, c):
        pltpu.async_copy(h_hbm.at[colv.at[q]], bufs[c], gsem[c])

    def _gather_wait(q, c):
        pltpu.make_async_copy(h_hbm.at[colv.at[q]], bufs[c], gsem[c]).wait()

    def _scat_start(q, c):
        pltpu.async_copy(bufs[c], acc.at[rowv.at[q]], ssem[c], add=True)

    def _scat_wait(q, c):
        pltpu.make_async_copy(bufs[c], acc.at[rowv.at[q]], ssem[c]).wait()

    for g in range(6):
        _idx_start(g, g)

    def _zrow(i, carry):
        for f in range(_H // 16):
            b0[i, pl.ds(f * 16, 16)] = jnp.zeros((16,), jnp.float32)
        return carry
    lax.fori_loop(0, _CHUNK, _zrow, 0)
    for k in range(_RPT // _CHUNK):
        pltpu.sync_copy(b0, acc.at[pl.ds(sid * _RPT + k * _CHUNK, _CHUNK)])

    for g in range(2):
        _idx_wait(g, g)
        _gather_start(g, g)
    plsc.subcore_barrier()

    def _iter(i, carry):
        for b in range(_NIDX):
            g = i * _NIDX + b
            c = b % _NBUF
            _gather_wait(b, c)

            def _grp(g16, c2):
                v16 = valv[b, pl.ds(g16 * 16, 16)]
                for j in range(16):
                    e = g16 * 16 + j
                    bv = jnp.full((16,), v16[j], jnp.float32)
                    for f in range(_H // 16):
                        sl = (e, pl.ds(f * 16, 16))
                        bufs[c][sl] = bufs[c][sl] * bv
                return c2
            lax.fori_loop(0, _CHUNK // 16, _grp, 0)

            _scat_start(b, c)

            @pl.when(g >= 2)
            def _():
                _scat_wait((b - 2) % _NIDX, (b - 2) % _NBUF)

            @pl.when(g + 6 < _NCHUNK)
            def _():
                _idx_start(g + 6, (b + 6) % _NIDX)

            @pl.when(g + 2 < _NCHUNK)
            def _():
                _idx_wait(g + 2, (b + 2) % _NIDX)
                _gather_start((b + 2) % _NIDX, (b + 2) % _NBUF)
        return carry
    lax.fori_loop(0, _NCHUNK // _NIDX, _iter, 0)

    for g in (_NCHUNK - 2, _NCHUNK - 1):
        _scat_wait(g % _NIDX, g % _NBUF)

    plsc.subcore_barrier()
    pltpu.sync_copy(acc.at[pl.ds(sid * _RPT, _RPT)],
                    out_hbm.at[cid, pl.ds(sid * _RPT, _RPT)])


def _spmm_partials(h, col, row, val):
    mesh = plsc.VectorSubcoreMesh(core_axis_name="c", subcore_axis_name="s")
    k = functools.partial(
        pl.kernel, mesh=mesh,
        out_type=jax.ShapeDtypeStruct((_NC, _NP, _H), jnp.float32),
        scratch_types=[
            pltpu.VMEM((_NIDX, _CHUNK), jnp.int32),
            pltpu.VMEM((_NIDX, _CHUNK), jnp.int32),
            pltpu.VMEM((_NIDX, _CHUNK), jnp.float32),
            pltpu.VMEM((_CHUNK, _H), jnp.float32),
            pltpu.VMEM((_CHUNK, _H), jnp.float32),
            pltpu.VMEM((_CHUNK, _H), jnp.float32),
            pltpu.VMEM((_CHUNK, _H), jnp.float32),
            pltpu.VMEM_SHARED((_NP, _H), jnp.float32),
        ] + [pltpu.SemaphoreType.DMA] * 16,
    )(_spmm_sc_body)
    return k(h, col, row, val)


def _gelu(x):
    return 0.5 * x * (1.0 + lax.erf(x * (2.0 ** -0.5)))


def _lin1_body(x_ref, w_ref, b_ref, o_ref):
    o_ref[...] = (
        jnp.dot(x_ref[...], w_ref[...], preferred_element_type=jnp.float32)
        + b_ref[...]
    )


def _mid_body(p_ref, w_ref, b_ref, o_ref):
    s = _gelu(p_ref[0] + p_ref[1])
    o_ref[...] = (
        jnp.dot(s, w_ref[...], preferred_element_type=jnp.float32)
        + b_ref[...]
    )


def _fin_body(p_ref, x_ref, g_ref, bt_ref, o_ref):
    r = _gelu(p_ref[0] + p_ref[1]) + x_ref[...]
    mean = jnp.mean(r, axis=-1, keepdims=True)
    var = jnp.mean((r - mean) ** 2, axis=-1, keepdims=True)
    o_ref[...] = (r - mean) / jnp.sqrt(var + 1e-5) * g_ref[...] + bt_ref[...]


_BLK = 1000
_GRID = _N // _BLK


def _linear1(x, w_t, b):
    return pl.pallas_call(
        _lin1_body,
        grid=(_GRID,),
        in_specs=[
            pl.BlockSpec((_BLK, _D), lambda i: (i, 0)),
            pl.BlockSpec((_D, _H), lambda i: (0, 0)),
            pl.BlockSpec((1, _H), lambda i: (0, 0)),
        ],
        out_specs=pl.BlockSpec((_BLK, _H), lambda i: (i, 0)),
        out_shape=jax.ShapeDtypeStruct((_N, _H), jnp.float32),
    )(x, w_t, b)


def _mid(p, w_t, b):
    return pl.pallas_call(
        _mid_body,
        grid=(_GRID,),
        in_specs=[
            pl.BlockSpec((_NC, _BLK, _H), lambda i: (0, i, 0)),
            pl.BlockSpec((_H, _H), lambda i: (0, 0)),
            pl.BlockSpec((1, _H), lambda i: (0, 0)),
        ],
        out_specs=pl.BlockSpec((_BLK, _H), lambda i: (i, 0)),
        out_shape=jax.ShapeDtypeStruct((_N, _H), jnp.float32),
    )(p, w_t, b)


def _final(p, x, gamma, beta):
    return pl.pallas_call(
        _fin_body,
        grid=(_GRID,),
        in_specs=[
            pl.BlockSpec((_NC, _BLK, _H), lambda i: (0, i, 0)),
            pl.BlockSpec((_BLK, _H), lambda i: (i, 0)),
            pl.BlockSpec((1, _H), lambda i: (0, 0)),
            pl.BlockSpec((1, _H), lambda i: (0, 0)),
        ],
        out_specs=pl.BlockSpec((_BLK, _H), lambda i: (i, 0)),
        out_shape=jax.ShapeDtypeStruct((_N, _H), jnp.float32),
    )(p, x, gamma, beta)


def kernel(x, adj_indices, adj_values, W1, b1, W2, b2, ln_gamma, ln_beta):
    pad = _EPAD - _E
    row = jnp.pad(adj_indices[0], (0, pad))
    col = jnp.pad(adj_indices[1], (0, pad))
    val = jnp.pad(adj_values, (0, pad))

    t1 = _linear1(x, W1.T, b1.reshape(1, _H))
    p1 = _spmm_partials(t1, col, row, val)
    t2 = _mid(p1, W2.T, b2.reshape(1, _H))
    p2 = _spmm_partials(t2, col, row, val)
    return _final(p2, x[:, :_H], ln_gamma.reshape(1, _H),
                  ln_beta.reshape(1, _H))
